# baseline (speedup 1.0000x reference)
import jax
import jax.numpy as jnp
from jax import lax
from jax.experimental import pallas as pl
from jax.experimental.pallas import tpu as pltpu
from jax.experimental.pallas import tpu_sc as plsc

N_NODES = 50000
N_EDGES = 800000
IN1_DIM = 80
OUT_DIM = 240
OUT_PAD = 256
XREC = 128

NB = 80
CH = 128
T_TASKS = N_NODES // NB
NOFF = 640

INV_SQRT3 = 1.0 / (3.0 ** 0.5)
INV_SQRT2 = 1.0 / (2.0 ** 0.5)

NC = 2
NS = 16
NW = NC * NS


def _sc_body(x_hbm, rows_hbm, cols_hbm, wt_hbm, y_hbm, offs_hbm, z_hbm,
             acc_v, rows_v, cols_v, wt_v, y_v, x_v, offs_v):
    wid = lax.axis_index("s") * NC + lax.axis_index("c")

    pltpu.sync_copy(offs_hbm, offs_v)

    iota = lax.iota(jnp.int32, 16)
    zero16 = jnp.zeros((16,), jnp.float32)
    c_o1a = iota
    c_o1b = iota + 16
    c_o2 = iota + 32
    c_o3a = [3 * iota + (48 + c) for c in range(3)]
    c_o3b = [3 * iota + (96 + c) for c in range(3)]
    c_o4 = [3 * iota + (144 + c) for c in range(3)]
    c_o5 = [3 * iota + (192 + c) for c in range(3)]

    def edge_body(j, carry):
        n0 = carry
        jv = jnp.full((16,), j, jnp.int32)
        r = plsc.load_gather(rows_v, [jv]) - n0

        x0a = plsc.load_gather(x_v, [jv, iota])
        x0b = plsc.load_gather(x_v, [jv, iota + 16])
        x1 = [plsc.load_gather(x_v, [jv, 3 * iota + (32 + c)]) for c in range(3)]

        y0 = plsc.load_gather(y_v, [jv])
        y1 = [plsc.load_gather(y_v, [jv + (1 + c) * CH]) for c in range(3)]

        w1a = plsc.load_gather(wt_v, [iota, jv])
        w1b = plsc.load_gather(wt_v, [iota + 16, jv])
        w2 = plsc.load_gather(wt_v, [iota + 32, jv])
        w3a = plsc.load_gather(wt_v, [iota + 48, jv])
        w3b = plsc.load_gather(wt_v, [iota + 64, jv])
        w4 = plsc.load_gather(wt_v, [iota + 80, jv])
        w5 = plsc.load_gather(wt_v, [iota + 96, jv])

        plsc.addupdate_scatter(acc_v, [c_o1a, r], w1a * (x0a * y0))
        plsc.addupdate_scatter(acc_v, [c_o1b, r], w1b * (x0b * y0))
        dot = x1[0] * y1[0] + x1[1] * y1[1] + x1[2] * y1[2]
        plsc.addupdate_scatter(acc_v, [c_o2, r], w2 * (dot * INV_SQRT3))
        t3a = w3a * x0a
        t3b = w3b * x0b
        for c in range(3):
            plsc.addupdate_scatter(acc_v, [c_o3a[c], r], t3a * y1[c])
            plsc.addupdate_scatter(acc_v, [c_o3b[c], r], t3b * y1[c])
        t4 = w4 * y0
        for c in range(3):
            plsc.addupdate_scatter(acc_v, [c_o4[c], r], t4 * x1[c])
        w5s = w5 * INV_SQRT2
        cr = [x1[1] * y1[2] - x1[2] * y1[1],
              x1[2] * y1[0] - x1[0] * y1[2],
              x1[0] * y1[1] - x1[1] * y1[0]]
        for c in range(3):
            plsc.addupdate_scatter(acc_v, [c_o5[c], r], w5s * cr[c])
        return carry

    def chunk_body(c, carry):
        n0, e0, e1, e0a = carry
        bnom = e0a + c * CH
        base = jnp.minimum(bnom, N_EDGES - CH)
        pltpu.sync_copy(rows_hbm.at[pl.ds(base, CH)], rows_v)
        pltpu.sync_copy(cols_hbm.at[pl.ds(base, CH)], cols_v)
        pltpu.sync_copy(wt_hbm.at[:, pl.ds(base, CH)], wt_v)
        for comp in range(4):
            pltpu.sync_copy(y_hbm.at[pl.ds(comp * N_EDGES + base, CH)],
                            y_v.at[pl.ds(comp * CH, CH)])
        pltpu.sync_copy(x_hbm.at[cols_v], x_v)
        jlo = jnp.maximum(e0, bnom) - base
        jhi = jnp.minimum(e1, bnom + CH) - base
        lax.fori_loop(jlo, jhi, edge_body, n0, unroll=False)
        return carry

    def zero_body(i, _):
        for k in range(NB // 16):
            acc_v[i, pl.ds(16 * k, 16)] = zero16
        return 0

    def task_body(i, _):
        t = wid + i * NW
        n0 = t * NB
        ev = offs_v[pl.ds(t, 16)]
        e0 = ev[0]
        e1 = ev[1]
        e0a = (e0 // 8) * 8
        lax.fori_loop(0, OUT_DIM, zero_body, 0, unroll=False)
        nchunks = (e1 - e0a + CH - 1) // CH
        lax.fori_loop(0, nchunks, chunk_body, (n0, e0, e1, e0a), unroll=False)
        pltpu.sync_copy(acc_v, z_hbm.at[:, pl.ds(n0, NB)])
        return 0

    ntasks = (T_TASKS - wid + NW - 1) // NW
    lax.fori_loop(0, ntasks, task_body, 0, unroll=False)


@jax.jit
def _tp_conv(X, Y, W, rows, cols):
    x_p = jnp.concatenate(
        [X, jnp.zeros((N_NODES, XREC - IN1_DIM), jnp.float32)], axis=1)
    wt = W.T
    yf = Y.T.reshape(-1)
    bounds = jnp.arange(0, NOFF, dtype=jnp.int32) * NB
    offs = jnp.searchsorted(rows, bounds, side="left").astype(jnp.int32)
    offs = jnp.minimum(offs, N_EDGES)

    mesh = plsc.VectorSubcoreMesh(core_axis_name="c", subcore_axis_name="s")
    run = pl.kernel(
        _sc_body,
        out_type=jax.ShapeDtypeStruct((OUT_DIM, N_NODES), jnp.float32),
        mesh=mesh,
        compiler_params=pltpu.CompilerParams(
            needs_layout_passes=False, use_tc_tiling_on_sc=False),
        scratch_types=[
            pltpu.VMEM((OUT_DIM, NB), jnp.float32),
            pltpu.VMEM((CH,), jnp.int32),
            pltpu.VMEM((CH,), jnp.int32),
            pltpu.VMEM((112, CH), jnp.float32),
            pltpu.VMEM((4 * CH,), jnp.float32),
            pltpu.VMEM((CH, XREC), jnp.float32),
            pltpu.VMEM((NOFF,), jnp.int32),
        ],
    )
    z = run(x_p, rows, cols, wt, yf, offs)
    return z.T


def kernel(X, Y, W, rows, cols):
    return _tp_conv(X, Y, W, rows, cols)


# speedup vs baseline: 2.8401x; 2.8401x over previous
import jax
import jax.numpy as jnp
from jax import lax
from jax.experimental import pallas as pl
from jax.experimental.pallas import tpu as pltpu
from jax.experimental.pallas import tpu_sc as plsc

N_NODES = 50000
N_EDGES = 800000
IN1_DIM = 80
W_NUMEL = 112
OUT_DIM = 240
XREC = 128
REC = 128

NB = 80
CH = 128
T_TASKS = N_NODES // NB
NOFF = 640

INV_SQRT3 = 1.0 / (3.0 ** 0.5)
INV_SQRT2 = 1.0 / (2.0 ** 0.5)

NC = 2
NS = 16
NW = NC * NS


def _sc_body(x_hbm, rows_hbm, cols_hbm, w_hbm, offs_hbm, z_hbm,
             acc_v, rows_v, cols_v, w_v, x_v, offs_v):
    wid = lax.axis_index("s") * NC + lax.axis_index("c")

    pltpu.sync_copy(offs_hbm, offs_v)

    iota = lax.iota(jnp.int32, 16)
    zero16 = jnp.zeros((16,), jnp.float32)
    c_o1a = iota
    c_o1b = iota + 16
    c_o2 = iota + 32
    c_o3a = [3 * iota + (48 + c) for c in range(3)]
    c_o3b = [3 * iota + (96 + c) for c in range(3)]
    c_o4 = [3 * iota + (144 + c) for c in range(3)]
    c_o5 = [3 * iota + (192 + c) for c in range(3)]

    def edge_body(j, carry):
        n0 = carry
        jv = jnp.full((16,), j, jnp.int32)
        r = plsc.load_gather(rows_v, [jv]) - n0

        x0a = plsc.load_gather(x_v, [jv, iota])
        x0b = plsc.load_gather(x_v, [jv, iota + 16])
        x1 = [plsc.load_gather(x_v, [jv, 3 * iota + (32 + c)]) for c in range(3)]

        y0 = plsc.load_gather(w_v, [jv, jnp.full((16,), 112, jnp.int32)])
        y1 = [plsc.load_gather(w_v, [jv, jnp.full((16,), 113 + c, jnp.int32)])
              for c in range(3)]

        w1a = plsc.load_gather(w_v, [jv, iota])
        w1b = plsc.load_gather(w_v, [jv, iota + 16])
        w2 = plsc.load_gather(w_v, [jv, iota + 32])
        w3a = plsc.load_gather(w_v, [jv, iota + 48])
        w3b = plsc.load_gather(w_v, [jv, iota + 64])
        w4 = plsc.load_gather(w_v, [jv, iota + 80])
        w5 = plsc.load_gather(w_v, [jv, iota + 96])

        plsc.addupdate_scatter(acc_v, [c_o1a, r], w1a * (x0a * y0))
        plsc.addupdate_scatter(acc_v, [c_o1b, r], w1b * (x0b * y0))
        dot = x1[0] * y1[0] + x1[1] * y1[1] + x1[2] * y1[2]
        plsc.addupdate_scatter(acc_v, [c_o2, r], w2 * (dot * INV_SQRT3))
        t3a = w3a * x0a
        t3b = w3b * x0b
        for c in range(3):
            plsc.addupdate_scatter(acc_v, [c_o3a[c], r], t3a * y1[c])
            plsc.addupdate_scatter(acc_v, [c_o3b[c], r], t3b * y1[c])
        t4 = w4 * y0
        for c in range(3):
            plsc.addupdate_scatter(acc_v, [c_o4[c], r], t4 * x1[c])
        w5s = w5 * INV_SQRT2
        cr = [x1[1] * y1[2] - x1[2] * y1[1],
              x1[2] * y1[0] - x1[0] * y1[2],
              x1[0] * y1[1] - x1[1] * y1[0]]
        for c in range(3):
            plsc.addupdate_scatter(acc_v, [c_o5[c], r], w5s * cr[c])
        return carry

    def chunk_body(c, carry):
        n0, e0, e1, e0a = carry
        bnom = e0a + c * CH
        base = jnp.minimum(bnom, N_EDGES - CH)
        pltpu.sync_copy(rows_hbm.at[pl.ds(base, CH)], rows_v)
        pltpu.sync_copy(cols_hbm.at[pl.ds(base, CH)], cols_v)
        pltpu.sync_copy(w_hbm.at[pl.ds(base, CH)], w_v)
        pltpu.sync_copy(x_hbm.at[cols_v], x_v)
        jlo = jnp.maximum(e0, bnom) - base
        jhi = jnp.minimum(e1, bnom + CH) - base
        lax.fori_loop(jlo, jhi, edge_body, n0, unroll=False)
        return carry

    def zero_body(i, _):
        for k in range(NB // 16):
            acc_v[i, pl.ds(16 * k, 16)] = zero16
        return 0

    def task_body(i, _):
        t = wid + i * NW
        n0 = t * NB
        ev = offs_v[pl.ds(t, 16)]
        e0 = ev[0]
        e1 = ev[1]
        e0a = (e0 // 8) * 8
        lax.fori_loop(0, OUT_DIM, zero_body, 0, unroll=False)
        nchunks = (e1 - e0a + CH - 1) // CH
        lax.fori_loop(0, nchunks, chunk_body, (n0, e0, e1, e0a), unroll=False)
        pltpu.sync_copy(acc_v, z_hbm.at[:, pl.ds(n0, NB)])
        return 0

    ntasks = (T_TASKS - wid + NW - 1) // NW
    lax.fori_loop(0, ntasks, task_body, 0, unroll=False)


_PACK_E = 640


def _packw_body(wt_ref, yt_ref, o_ref):
    eye_w = jnp.float32(
        lax.broadcasted_iota(jnp.int32, (W_NUMEL, W_NUMEL), 0)
        == lax.broadcasted_iota(jnp.int32, (W_NUMEL, W_NUMEL), 1))
    eye_y = jnp.float32(
        lax.broadcasted_iota(jnp.int32, (4, 4), 0)
        == lax.broadcasted_iota(jnp.int32, (4, 4), 1))
    wt = lax.dot_general(wt_ref[...], eye_w, (((0,), (0,)), ((), ())),
                         preferred_element_type=jnp.float32)
    yt = lax.dot_general(yt_ref[...], eye_y, (((0,), (0,)), ((), ())),
                         preferred_element_type=jnp.float32)
    o_ref[...] = jnp.concatenate(
        [wt, yt, jnp.zeros((_PACK_E, REC - W_NUMEL - 4), jnp.float32)], axis=1)


_packw = pl.pallas_call(
    _packw_body,
    grid=(N_EDGES // _PACK_E,),
    in_specs=[
        pl.BlockSpec((W_NUMEL, _PACK_E), lambda i: (0, i)),
        pl.BlockSpec((4, _PACK_E), lambda i: (0, i)),
    ],
    out_specs=pl.BlockSpec((_PACK_E, REC), lambda i: (i, 0)),
    out_shape=jax.ShapeDtypeStruct((N_EDGES, REC), jnp.float32),
)


@jax.jit
def _tp_conv(X, Y, W, rows, cols):
    x_p = jnp.concatenate(
        [X, jnp.zeros((N_NODES, XREC - IN1_DIM), jnp.float32)], axis=1)
    w_p = _packw(W.T, Y.T)
    bounds = jnp.arange(0, NOFF, dtype=jnp.int32) * NB
    offs = jnp.searchsorted(rows, bounds, side="left").astype(jnp.int32)
    offs = jnp.minimum(offs, N_EDGES)

    mesh = plsc.VectorSubcoreMesh(core_axis_name="c", subcore_axis_name="s")
    run = pl.kernel(
        _sc_body,
        out_type=jax.ShapeDtypeStruct((OUT_DIM, N_NODES), jnp.float32),
        mesh=mesh,
        compiler_params=pltpu.CompilerParams(
            needs_layout_passes=False, use_tc_tiling_on_sc=False),
        scratch_types=[
            pltpu.VMEM((OUT_DIM, NB), jnp.float32),
            pltpu.VMEM((CH,), jnp.int32),
            pltpu.VMEM((CH,), jnp.int32),
            pltpu.VMEM((CH, REC), jnp.float32),
            pltpu.VMEM((CH, XREC), jnp.float32),
            pltpu.VMEM((NOFF,), jnp.int32),
        ],
    )
    z = run(x_p, rows, cols, w_p, offs)
    return z.T


def kernel(X, Y, W, rows, cols):
    return _tp_conv(X, Y, W, rows, cols)


# speedup vs baseline: 4.6486x; 1.6368x over previous
import jax
import jax.numpy as jnp
from jax import lax
from jax.experimental import pallas as pl
from jax.experimental.pallas import tpu as pltpu
from jax.experimental.pallas import tpu_sc as plsc

N_NODES = 50000
N_EDGES = 800000
IN1_DIM = 80
W_NUMEL = 112
OUT_DIM = 240
OUT_PAD = 256
XREC = 128
REC = 128

NB = 80
CH = 128
T_TASKS = N_NODES // NB
NOFF = 640

INV_SQRT3 = 1.0 / (3.0 ** 0.5)
INV_SQRT2 = 1.0 / (2.0 ** 0.5)

NC = 2
NS = 16
NW = NC * NS


def _sc_body(x_hbm, rows_hbm, cols_hbm, w_hbm, offs_hbm, z_hbm,
             acc_v, rows_v, cols_v, w_v, x_v, offs_v):
    wid = lax.axis_index("s") * NC + lax.axis_index("c")

    pltpu.sync_copy(offs_hbm, offs_v)

    iota = lax.iota(jnp.int32, 16)
    zero16 = jnp.zeros((16,), jnp.float32)
    c_o1a = iota
    c_o1b = iota + 16
    c_o2 = iota + 32
    c_o3a = [3 * iota + (48 + c) for c in range(3)]
    c_o3b = [3 * iota + (96 + c) for c in range(3)]
    c_o4 = [3 * iota + (144 + c) for c in range(3)]
    c_o5 = [3 * iota + (192 + c) for c in range(3)]

    def edge_body(j, carry):
        n0 = carry
        jv = jnp.full((16,), j, jnp.int32)
        r = plsc.load_gather(rows_v, [jv]) - n0

        x0a = plsc.load_gather(x_v, [jv, iota])
        x0b = plsc.load_gather(x_v, [jv, iota + 16])
        x1 = [plsc.load_gather(x_v, [jv, 3 * iota + (32 + c)]) for c in range(3)]

        y0 = plsc.load_gather(w_v, [jv, jnp.full((16,), 112, jnp.int32)])
        y1 = [plsc.load_gather(w_v, [jv, jnp.full((16,), 113 + c, jnp.int32)])
              for c in range(3)]

        w1a = plsc.load_gather(w_v, [jv, iota])
        w1b = plsc.load_gather(w_v, [jv, iota + 16])
        w2 = plsc.load_gather(w_v, [jv, iota + 32])
        w3a = plsc.load_gather(w_v, [jv, iota + 48])
        w3b = plsc.load_gather(w_v, [jv, iota + 64])
        w4 = plsc.load_gather(w_v, [jv, iota + 80])
        w5 = plsc.load_gather(w_v, [jv, iota + 96])

        plsc.addupdate_scatter(acc_v, [r, c_o1a], w1a * (x0a * y0))
        plsc.addupdate_scatter(acc_v, [r, c_o1b], w1b * (x0b * y0))
        dot = x1[0] * y1[0] + x1[1] * y1[1] + x1[2] * y1[2]
        plsc.addupdate_scatter(acc_v, [r, c_o2], w2 * (dot * INV_SQRT3))
        t3a = w3a * x0a
        t3b = w3b * x0b
        for c in range(3):
            plsc.addupdate_scatter(acc_v, [r, c_o3a[c]], t3a * y1[c])
            plsc.addupdate_scatter(acc_v, [r, c_o3b[c]], t3b * y1[c])
        t4 = w4 * y0
        for c in range(3):
            plsc.addupdate_scatter(acc_v, [r, c_o4[c]], t4 * x1[c])
        w5s = w5 * INV_SQRT2
        cr = [x1[1] * y1[2] - x1[2] * y1[1],
              x1[2] * y1[0] - x1[0] * y1[2],
              x1[0] * y1[1] - x1[1] * y1[0]]
        for c in range(3):
            plsc.addupdate_scatter(acc_v, [r, c_o5[c]], w5s * cr[c])
        return carry

    def chunk_body(c, carry):
        n0, e0, e1, e0a = carry
        bnom = e0a + c * CH
        base = jnp.minimum(bnom, N_EDGES - CH)
        pltpu.sync_copy(rows_hbm.at[pl.ds(base, CH)], rows_v)
        pltpu.sync_copy(cols_hbm.at[pl.ds(base, CH)], cols_v)
        pltpu.sync_copy(w_hbm.at[pl.ds(base, CH)], w_v)
        pltpu.sync_copy(x_hbm.at[cols_v], x_v)
        jlo = jnp.maximum(e0, bnom) - base
        jhi = jnp.minimum(e1, bnom + CH) - base
        lax.fori_loop(jlo, jhi, edge_body, n0, unroll=False)
        return carry

    def zero_body(i, _):
        for k in range(OUT_PAD // 16):
            acc_v[i, pl.ds(16 * k, 16)] = zero16
        return 0

    def task_body(i, _):
        t = wid + i * NW
        n0 = t * NB
        ev = offs_v[pl.ds(t, 16)]
        e0 = ev[0]
        e1 = ev[1]
        e0a = (e0 // 8) * 8
        lax.fori_loop(0, NB, zero_body, 0, unroll=False)
        nchunks = (e1 - e0a + CH - 1) // CH
        lax.fori_loop(0, nchunks, chunk_body, (n0, e0, e1, e0a), unroll=False)
        pltpu.sync_copy(acc_v, z_hbm.at[pl.ds(n0, NB)])
        return 0

    ntasks = (T_TASKS - wid + NW - 1) // NW
    lax.fori_loop(0, ntasks, task_body, 0, unroll=False)


def _eye(n):
    return jnp.float32(
        lax.broadcasted_iota(jnp.int32, (n, n), 0)
        == lax.broadcasted_iota(jnp.int32, (n, n), 1))


_PACK_E = 6400


def _packw_body(wt_ref, yt_ref, o_ref):
    wt = lax.dot_general(wt_ref[...], _eye(W_NUMEL), (((0,), (0,)), ((), ())),
                         preferred_element_type=jnp.float32,
                         precision=lax.Precision.HIGHEST)
    yt = lax.dot_general(yt_ref[...], _eye(4), (((0,), (0,)), ((), ())),
                         preferred_element_type=jnp.float32,
                         precision=lax.Precision.HIGHEST)
    o_ref[...] = jnp.concatenate(
        [wt, yt, jnp.zeros((_PACK_E, REC - W_NUMEL - 4), jnp.float32)], axis=1)


_packw = pl.pallas_call(
    _packw_body,
    grid=(N_EDGES // _PACK_E,),
    in_specs=[
        pl.BlockSpec((W_NUMEL, _PACK_E), lambda i: (0, i)),
        pl.BlockSpec((4, _PACK_E), lambda i: (0, i)),
    ],
    out_specs=pl.BlockSpec((_PACK_E, REC), lambda i: (i, 0)),
    out_shape=jax.ShapeDtypeStruct((N_EDGES, REC), jnp.float32),
)

_UNPACK_N = 512
_N_PADDED = 50176


def _unpack_body(z_ref, o_ref):
    o_ref[...] = lax.dot_general(
        _eye(OUT_DIM), z_ref[:, :OUT_DIM], (((1,), (1,)), ((), ())),
        preferred_element_type=jnp.float32,
        precision=lax.Precision.HIGHEST)


_unpack = pl.pallas_call(
    _unpack_body,
    grid=(_N_PADDED // _UNPACK_N,),
    in_specs=[pl.BlockSpec((_UNPACK_N, OUT_PAD), lambda i: (i, 0))],
    out_specs=pl.BlockSpec((OUT_DIM, _UNPACK_N), lambda i: (0, i)),
    out_shape=jax.ShapeDtypeStruct((OUT_DIM, _N_PADDED), jnp.float32),
)


@jax.jit
def _tp_conv(X, Y, W, rows, cols):
    x_p = jnp.concatenate(
        [X, jnp.zeros((N_NODES, XREC - IN1_DIM), jnp.float32)], axis=1)
    w_p = _packw(W.T, Y.T)
    bounds = jnp.arange(0, NOFF, dtype=jnp.int32) * NB
    offs = jnp.searchsorted(rows, bounds, side="left").astype(jnp.int32)
    offs = jnp.minimum(offs, N_EDGES)

    mesh = plsc.VectorSubcoreMesh(core_axis_name="c", subcore_axis_name="s")
    run = pl.kernel(
        _sc_body,
        out_type=jax.ShapeDtypeStruct((N_NODES, OUT_PAD), jnp.float32),
        mesh=mesh,
        compiler_params=pltpu.CompilerParams(
            needs_layout_passes=False, use_tc_tiling_on_sc=False),
        scratch_types=[
            pltpu.VMEM((NB, OUT_PAD), jnp.float32),
            pltpu.VMEM((CH,), jnp.int32),
            pltpu.VMEM((CH,), jnp.int32),
            pltpu.VMEM((CH, REC), jnp.float32),
            pltpu.VMEM((CH, XREC), jnp.float32),
            pltpu.VMEM((NOFF,), jnp.int32),
        ],
    )
    z = run(x_p, rows, cols, w_p, offs)
    return _unpack(z)[:, :N_NODES].T


def kernel(X, Y, W, rows, cols):
    return _tp_conv(X, Y, W, rows, cols)


# speedup vs baseline: 5.1943x; 1.1174x over previous
import jax
import jax.numpy as jnp
from jax import lax
from jax.experimental import pallas as pl
from jax.experimental.pallas import tpu as pltpu
from jax.experimental.pallas import tpu_sc as plsc

N_NODES = 50000
N_EDGES = 800000
IN1_DIM = 80
W_NUMEL = 112
OUT_DIM = 240
OUT_PAD = 256
XREC = 128
REC = 128

NB = 80
CH = 128
T_TASKS = N_NODES // NB
NOFF = 640

INV_SQRT3 = 1.0 / (3.0 ** 0.5)
INV_SQRT2 = 1.0 / (2.0 ** 0.5)

NC = 2
NS = 16
NW = NC * NS


def _sc_body(x_hbm, rows_hbm, cols_hbm, w_hbm, offs_hbm, z_hbm,
             acc_v, rows_v, cols_v, w_v, x_v, offs_v):
    wid = lax.axis_index("s") * NC + lax.axis_index("c")

    pltpu.sync_copy(offs_hbm, offs_v)

    iota = lax.iota(jnp.int32, 16)
    zero16 = jnp.zeros((16,), jnp.float32)
    c_o1a = iota
    c_o1b = iota + 16
    c_o2 = iota + 32
    c_o3a = [3 * iota + (48 + c) for c in range(3)]
    c_o3b = [3 * iota + (96 + c) for c in range(3)]
    c_o4 = [3 * iota + (144 + c) for c in range(3)]
    c_o5 = [3 * iota + (192 + c) for c in range(3)]

    def edge_body(j, carry):
        n0 = carry
        jv = jnp.full((16,), j, jnp.int32)
        r = plsc.load_gather(rows_v, [jv]) - n0

        x0a = plsc.load_gather(x_v, [jv, iota])
        x0b = plsc.load_gather(x_v, [jv, iota + 16])
        x1 = [plsc.load_gather(x_v, [jv, 3 * iota + (32 + c)]) for c in range(3)]

        y0 = plsc.load_gather(w_v, [jv, jnp.full((16,), 112, jnp.int32)])
        y1 = [plsc.load_gather(w_v, [jv, jnp.full((16,), 113 + c, jnp.int32)])
              for c in range(3)]

        w1a = plsc.load_gather(w_v, [jv, iota])
        w1b = plsc.load_gather(w_v, [jv, iota + 16])
        w2 = plsc.load_gather(w_v, [jv, iota + 32])
        w3a = plsc.load_gather(w_v, [jv, iota + 48])
        w3b = plsc.load_gather(w_v, [jv, iota + 64])
        w4 = plsc.load_gather(w_v, [jv, iota + 80])
        w5 = plsc.load_gather(w_v, [jv, iota + 96])

        plsc.addupdate_scatter(acc_v, [r, c_o1a], w1a * (x0a * y0))
        plsc.addupdate_scatter(acc_v, [r, c_o1b], w1b * (x0b * y0))
        dot = x1[0] * y1[0] + x1[1] * y1[1] + x1[2] * y1[2]
        plsc.addupdate_scatter(acc_v, [r, c_o2], w2 * (dot * INV_SQRT3))
        t3a = w3a * x0a
        t3b = w3b * x0b
        for c in range(3):
            plsc.addupdate_scatter(acc_v, [r, c_o3a[c]], t3a * y1[c])
            plsc.addupdate_scatter(acc_v, [r, c_o3b[c]], t3b * y1[c])
        t4 = w4 * y0
        for c in range(3):
            plsc.addupdate_scatter(acc_v, [r, c_o4[c]], t4 * x1[c])
        w5s = w5 * INV_SQRT2
        cr = [x1[1] * y1[2] - x1[2] * y1[1],
              x1[2] * y1[0] - x1[0] * y1[2],
              x1[0] * y1[1] - x1[1] * y1[0]]
        for c in range(3):
            plsc.addupdate_scatter(acc_v, [r, c_o5[c]], w5s * cr[c])
        return carry

    def chunk_body(c, carry):
        n0, e0, e1, e0a = carry
        bnom = e0a + c * CH
        base = jnp.minimum(bnom, N_EDGES - CH)
        pltpu.sync_copy(rows_hbm.at[pl.ds(base, CH)], rows_v)
        pltpu.sync_copy(cols_hbm.at[pl.ds(base, CH)], cols_v)
        pltpu.sync_copy(w_hbm.at[pl.ds(base, CH)], w_v)
        pltpu.sync_copy(x_hbm.at[cols_v], x_v)
        jlo = jnp.maximum(e0, bnom) - base
        jhi = jnp.minimum(e1, bnom + CH) - base
        lax.fori_loop(jlo, jhi, edge_body, n0, unroll=False)
        return carry

    def zero_body(i, _):
        for k in range(OUT_PAD // 16):
            acc_v[i, pl.ds(16 * k, 16)] = zero16
        return 0

    def task_body(i, _):
        t = wid + i * NW
        n0 = t * NB
        ev = offs_v[pl.ds(t, 16)]
        e0 = ev[0]
        e1 = ev[1]
        e0a = (e0 // 8) * 8
        lax.fori_loop(0, NB, zero_body, 0, unroll=False)
        nchunks = (e1 - e0a + CH - 1) // CH
        lax.fori_loop(0, nchunks, chunk_body, (n0, e0, e1, e0a), unroll=False)
        pltpu.sync_copy(acc_v, z_hbm.at[pl.ds(n0, NB)])
        return 0

    ntasks = (T_TASKS - wid + NW - 1) // NW
    lax.fori_loop(0, ntasks, task_body, 0, unroll=False)


def _eye(n):
    return jnp.float32(
        lax.broadcasted_iota(jnp.int32, (n, n), 0)
        == lax.broadcasted_iota(jnp.int32, (n, n), 1))


_PACK_E = 6400


def _packw_body(wt_ref, yt_ref, o_ref):
    wt = wt_ref[...].T
    yt = yt_ref[...].T
    o_ref[...] = jnp.concatenate(
        [wt, yt, jnp.zeros((_PACK_E, REC - W_NUMEL - 4), jnp.float32)], axis=1)


_packw = pl.pallas_call(
    _packw_body,
    grid=(N_EDGES // _PACK_E,),
    in_specs=[
        pl.BlockSpec((W_NUMEL, _PACK_E), lambda i: (0, i)),
        pl.BlockSpec((4, _PACK_E), lambda i: (0, i)),
    ],
    out_specs=pl.BlockSpec((_PACK_E, REC), lambda i: (i, 0)),
    out_shape=jax.ShapeDtypeStruct((N_EDGES, REC), jnp.float32),
)

_UNPACK_N = 512
_N_PADDED = 50176


def _unpack_body(z_ref, o_ref):
    o_ref[...] = z_ref[:, :OUT_DIM].T


_unpack = pl.pallas_call(
    _unpack_body,
    grid=(_N_PADDED // _UNPACK_N,),
    in_specs=[pl.BlockSpec((_UNPACK_N, OUT_PAD), lambda i: (i, 0))],
    out_specs=pl.BlockSpec((OUT_DIM, _UNPACK_N), lambda i: (0, i)),
    out_shape=jax.ShapeDtypeStruct((OUT_DIM, _N_PADDED), jnp.float32),
)


_XPACK_N = 6400
_N_XPAD = 51200


def _packx_body(xt_ref, o_ref):
    o_ref[...] = jnp.concatenate(
        [xt_ref[...].T, jnp.zeros((_XPACK_N, XREC - IN1_DIM), jnp.float32)],
        axis=1)


_packx = pl.pallas_call(
    _packx_body,
    grid=(_N_XPAD // _XPACK_N,),
    in_specs=[pl.BlockSpec((IN1_DIM, _XPACK_N), lambda i: (0, i))],
    out_specs=pl.BlockSpec((_XPACK_N, XREC), lambda i: (i, 0)),
    out_shape=jax.ShapeDtypeStruct((_N_XPAD, XREC), jnp.float32),
)


@jax.jit
def _tp_conv(X, Y, W, rows, cols):
    x_p = _packx(X.T)
    w_p = _packw(W.T, Y.T)
    bounds = jnp.arange(0, NOFF, dtype=jnp.int32) * NB
    offs = jnp.searchsorted(rows, bounds, side="left").astype(jnp.int32)
    offs = jnp.minimum(offs, N_EDGES)

    mesh = plsc.VectorSubcoreMesh(core_axis_name="c", subcore_axis_name="s")
    run = pl.kernel(
        _sc_body,
        out_type=jax.ShapeDtypeStruct((N_NODES, OUT_PAD), jnp.float32),
        mesh=mesh,
        compiler_params=pltpu.CompilerParams(
            needs_layout_passes=False, use_tc_tiling_on_sc=False),
        scratch_types=[
            pltpu.VMEM((NB, OUT_PAD), jnp.float32),
            pltpu.VMEM((CH,), jnp.int32),
            pltpu.VMEM((CH,), jnp.int32),
            pltpu.VMEM((CH, REC), jnp.float32),
            pltpu.VMEM((CH, XREC), jnp.float32),
            pltpu.VMEM((NOFF,), jnp.int32),
        ],
    )
    z = run(x_p, rows, cols, w_p, offs)
    return _unpack(z)[:, :N_NODES].T


def kernel(X, Y, W, rows, cols):
    return _tp_conv(X, Y, W, rows, cols)


# speedup vs baseline: 7.4556x; 1.4353x over previous
import jax
import jax.numpy as jnp
from jax import lax
from jax.experimental import pallas as pl
from jax.experimental.pallas import tpu as pltpu
from jax.experimental.pallas import tpu_sc as plsc

N_NODES = 50000
N_EDGES = 800000
IN1_DIM = 80
W_NUMEL = 112
OUT_DIM = 240
OUT_PAD = 256
XREC = 128
REC = 128

NB = 80
CH = 128
T_TASKS = N_NODES // NB
NOFF = 640

INV_SQRT3 = 1.0 / (3.0 ** 0.5)
INV_SQRT2 = 1.0 / (2.0 ** 0.5)

NC = 2
NS = 16
NW = NC * NS


def _sc_body(x_hbm, rows_hbm, cols_hbm, w_hbm, offs_hbm, z_hbm,
             acc_v,
             rows_b0, rows_b1, rows_b2,
             cols_b0, cols_b1, cols_b2,
             w_b0, w_b1, w_b2,
             x_b0, x_b1, x_b2,
             offs_v, semA, semB):
    rows_b = (rows_b0, rows_b1, rows_b2)
    cols_b = (cols_b0, cols_b1, cols_b2)
    w_b = (w_b0, w_b1, w_b2)
    x_b = (x_b0, x_b1, x_b2)
    wid = lax.axis_index("s") * NC + lax.axis_index("c")

    pltpu.sync_copy(offs_hbm, offs_v)

    iota = lax.iota(jnp.int32, 16)
    zero16 = jnp.zeros((16,), jnp.float32)
    c_o1a = iota
    c_o1b = iota + 16
    c_o2 = iota + 32
    c_o3a = [3 * iota + (48 + c) for c in range(3)]
    c_o3b = [3 * iota + (96 + c) for c in range(3)]
    c_o4 = [3 * iota + (144 + c) for c in range(3)]
    c_o5 = [3 * iota + (192 + c) for c in range(3)]

    def make_edge_body(k):
        rows_v, w_v, x_v = rows_b[k], w_b[k], x_b[k]
        return lambda j, carry: edge_step(j, carry, rows_v, w_v, x_v)

    def edge_step(j, carry, rows_v, w_v, x_v):
        n0 = carry
        jv = jnp.full((16,), j, jnp.int32)
        r = plsc.load_gather(rows_v, [jv]) - n0

        x0a = plsc.load_gather(x_v, [jv, iota])
        x0b = plsc.load_gather(x_v, [jv, iota + 16])
        x1 = [plsc.load_gather(x_v, [jv, 3 * iota + (32 + c)]) for c in range(3)]

        y0 = plsc.load_gather(w_v, [jv, jnp.full((16,), 112, jnp.int32)])
        y1 = [plsc.load_gather(w_v, [jv, jnp.full((16,), 113 + c, jnp.int32)])
              for c in range(3)]

        w1a = plsc.load_gather(w_v, [jv, iota])
        w1b = plsc.load_gather(w_v, [jv, iota + 16])
        w2 = plsc.load_gather(w_v, [jv, iota + 32])
        w3a = plsc.load_gather(w_v, [jv, iota + 48])
        w3b = plsc.load_gather(w_v, [jv, iota + 64])
        w4 = plsc.load_gather(w_v, [jv, iota + 80])
        w5 = plsc.load_gather(w_v, [jv, iota + 96])

        plsc.addupdate_scatter(acc_v, [r, c_o1a], w1a * (x0a * y0))
        plsc.addupdate_scatter(acc_v, [r, c_o1b], w1b * (x0b * y0))
        dot = x1[0] * y1[0] + x1[1] * y1[1] + x1[2] * y1[2]
        plsc.addupdate_scatter(acc_v, [r, c_o2], w2 * (dot * INV_SQRT3))
        t3a = w3a * x0a
        t3b = w3b * x0b
        for c in range(3):
            plsc.addupdate_scatter(acc_v, [r, c_o3a[c]], t3a * y1[c])
            plsc.addupdate_scatter(acc_v, [r, c_o3b[c]], t3b * y1[c])
        t4 = w4 * y0
        for c in range(3):
            plsc.addupdate_scatter(acc_v, [r, c_o4[c]], t4 * x1[c])
        w5s = w5 * INV_SQRT2
        cr = [x1[1] * y1[2] - x1[2] * y1[1],
              x1[2] * y1[0] - x1[0] * y1[2],
              x1[0] * y1[1] - x1[1] * y1[0]]
        for c in range(3):
            plsc.addupdate_scatter(acc_v, [r, c_o5[c]], w5s * cr[c])
        return carry

    def zero_body(i, _):
        for k in range(OUT_PAD // 16):
            acc_v[i, pl.ds(16 * k, 16)] = zero16
        return 0

    def task_body(i, _):
        t = wid + i * NW
        n0 = t * NB
        ev = offs_v[pl.ds(t, 16)]
        e0 = ev[0]
        e1 = ev[1]
        e0a = (e0 // 8) * 8
        nch = (e1 - e0a + CH - 1) // CH

        def base_of(c):
            return jnp.minimum(e0a + c * CH, N_EDGES - CH)

        def startA(k, c):
            b = base_of(c)
            pltpu.async_copy(rows_hbm.at[pl.ds(b, CH)], rows_b[k], semA.at[k])
            pltpu.async_copy(cols_hbm.at[pl.ds(b, CH)], cols_b[k], semA.at[k])
            pltpu.async_copy(w_hbm.at[pl.ds(b, CH)], w_b[k], semA.at[k])

        def waitA(k, c):
            b = base_of(c)
            pltpu.make_async_copy(
                rows_hbm.at[pl.ds(b, CH)], rows_b[k], semA.at[k]).wait()
            pltpu.make_async_copy(
                cols_hbm.at[pl.ds(b, CH)], cols_b[k], semA.at[k]).wait()
            pltpu.make_async_copy(
                w_hbm.at[pl.ds(b, CH)], w_b[k], semA.at[k]).wait()

        def startB(k):
            pltpu.async_copy(x_hbm.at[cols_b[k]], x_b[k], semB.at[k])

        def waitB(k):
            pltpu.make_async_copy(x_hbm.at[cols_b[k]], x_b[k], semB.at[k]).wait()

        @pl.when(nch > 0)
        def _():
            startA(0, 0)

        @pl.when(nch > 1)
        def _():
            startA(1, 1)

        lax.fori_loop(0, NB, zero_body, 0, unroll=False)

        @pl.when(nch > 0)
        def _():
            waitA(0, 0)
            startB(0)

        def group_body(g, _):
            for k in range(3):
                c = 3 * g + k

                @pl.when(c < nch)
                def _(c=c, k=k):
                    @pl.when(c + 1 < nch)
                    def _():
                        waitA((k + 1) % 3, c + 1)
                        startB((k + 1) % 3)

                    @pl.when(c + 2 < nch)
                    def _():
                        startA((k + 2) % 3, c + 2)

                    waitB(k)
                    bnom = e0a + c * CH
                    b = base_of(c)
                    jlo = jnp.maximum(e0, bnom) - b
                    jhi = jnp.minimum(e1, bnom + CH) - b
                    lax.fori_loop(jlo, jhi, make_edge_body(k), n0,
                                  unroll=False)
            return 0

        lax.fori_loop(0, (nch + 2) // 3, group_body, 0, unroll=False)
        pltpu.sync_copy(acc_v, z_hbm.at[pl.ds(n0, NB)])
        return 0

    ntasks = (T_TASKS - wid + NW - 1) // NW
    lax.fori_loop(0, ntasks, task_body, 0, unroll=False)


def _eye(n):
    return jnp.float32(
        lax.broadcasted_iota(jnp.int32, (n, n), 0)
        == lax.broadcasted_iota(jnp.int32, (n, n), 1))


_PACK_E = 6400


def _packw_body(wt_ref, yt_ref, o_ref):
    wt = wt_ref[...].T
    yt = yt_ref[...].T
    o_ref[...] = jnp.concatenate(
        [wt, yt, jnp.zeros((_PACK_E, REC - W_NUMEL - 4), jnp.float32)], axis=1)


_packw = pl.pallas_call(
    _packw_body,
    grid=(N_EDGES // _PACK_E,),
    in_specs=[
        pl.BlockSpec((W_NUMEL, _PACK_E), lambda i: (0, i)),
        pl.BlockSpec((4, _PACK_E), lambda i: (0, i)),
    ],
    out_specs=pl.BlockSpec((_PACK_E, REC), lambda i: (i, 0)),
    out_shape=jax.ShapeDtypeStruct((N_EDGES, REC), jnp.float32),
)

_UNPACK_N = 512
_N_PADDED = 50176


def _unpack_body(z_ref, o_ref):
    o_ref[...] = z_ref[:, :OUT_DIM].T


_unpack = pl.pallas_call(
    _unpack_body,
    grid=(_N_PADDED // _UNPACK_N,),
    in_specs=[pl.BlockSpec((_UNPACK_N, OUT_PAD), lambda i: (i, 0))],
    out_specs=pl.BlockSpec((OUT_DIM, _UNPACK_N), lambda i: (0, i)),
    out_shape=jax.ShapeDtypeStruct((OUT_DIM, _N_PADDED), jnp.float32),
)


_XPACK_N = 6400
_N_XPAD = 51200


def _packx_body(xt_ref, o_ref):
    o_ref[...] = jnp.concatenate(
        [xt_ref[...].T, jnp.zeros((_XPACK_N, XREC - IN1_DIM), jnp.float32)],
        axis=1)


_packx = pl.pallas_call(
    _packx_body,
    grid=(_N_XPAD // _XPACK_N,),
    in_specs=[pl.BlockSpec((IN1_DIM, _XPACK_N), lambda i: (0, i))],
    out_specs=pl.BlockSpec((_XPACK_N, XREC), lambda i: (i, 0)),
    out_shape=jax.ShapeDtypeStruct((_N_XPAD, XREC), jnp.float32),
)


@jax.jit
def _tp_conv(X, Y, W, rows, cols):
    x_p = _packx(X.T)
    w_p = _packw(W.T, Y.T)
    bounds = jnp.arange(0, NOFF, dtype=jnp.int32) * NB
    offs = jnp.searchsorted(rows, bounds, side="left").astype(jnp.int32)
    offs = jnp.minimum(offs, N_EDGES)

    mesh = plsc.VectorSubcoreMesh(core_axis_name="c", subcore_axis_name="s")
    run = pl.kernel(
        _sc_body,
        out_type=jax.ShapeDtypeStruct((N_NODES, OUT_PAD), jnp.float32),
        mesh=mesh,
        compiler_params=pltpu.CompilerParams(
            needs_layout_passes=False, use_tc_tiling_on_sc=False),
        scratch_types=(
            [pltpu.VMEM((NB, OUT_PAD), jnp.float32)]
            + [pltpu.VMEM((CH,), jnp.int32)] * 3
            + [pltpu.VMEM((CH,), jnp.int32)] * 3
            + [pltpu.VMEM((CH, REC), jnp.float32)] * 3
            + [pltpu.VMEM((CH, XREC), jnp.float32)] * 3
            + [pltpu.VMEM((NOFF,), jnp.int32),
               pltpu.SemaphoreType.DMA((3,)),
               pltpu.SemaphoreType.DMA((3,))]
        ),
    )
    z = run(x_p, rows, cols, w_p, offs)
    return _unpack(z)[:, :N_NODES].T


def kernel(X, Y, W, rows, cols):
    return _tp_conv(X, Y, W, rows, cols)


# speedup vs baseline: 7.5589x; 1.0139x over previous
import jax
import jax.numpy as jnp
from jax import lax
from jax.experimental import pallas as pl
from jax.experimental.pallas import tpu as pltpu
from jax.experimental.pallas import tpu_sc as plsc

N_NODES = 50000
N_EDGES = 800000
IN1_DIM = 80
W_NUMEL = 112
OUT_DIM = 240
OUT_PAD = 256
XREC = 128
REC = 128

NB = 80
CH = 128
T_TASKS = N_NODES // NB
NOFF = 640

INV_SQRT3 = 1.0 / (3.0 ** 0.5)
INV_SQRT2 = 1.0 / (2.0 ** 0.5)

NC = 2
NS = 16
NW = NC * NS


def _sc_body(x_hbm, rows_hbm, cols_hbm, w_hbm, offs_hbm, z_hbm,
             acc_v,
             rows_b0, rows_b1, rows_b2,
             cols_b0, cols_b1, cols_b2,
             w_b0, w_b1, w_b2,
             x_b0, x_b1, x_b2,
             offs_v, semA, semB):
    rows_b = (rows_b0, rows_b1, rows_b2)
    cols_b = (cols_b0, cols_b1, cols_b2)
    w_b = (w_b0, w_b1, w_b2)
    x_b = (x_b0, x_b1, x_b2)
    wid = lax.axis_index("s") * NC + lax.axis_index("c")

    pltpu.sync_copy(offs_hbm, offs_v)

    iota = lax.iota(jnp.int32, 16)
    zero16 = jnp.zeros((16,), jnp.float32)
    c_o1a = iota
    c_o1b = iota + 16
    c_o2 = iota + 32
    c_o3a = [3 * iota + (48 + c) for c in range(3)]
    c_o3b = [3 * iota + (96 + c) for c in range(3)]
    c_o4 = [3 * iota + (144 + c) for c in range(3)]
    c_o5 = [3 * iota + (192 + c) for c in range(3)]

    def make_edge_body(k):
        rows_v, w_v, x_v = rows_b[k], w_b[k], x_b[k]
        return lambda j, carry: edge_step(j, carry, rows_v, w_v, x_v)

    def edge_step(j, carry, rows_v, w_v, x_v):
        n0 = carry
        jv = jnp.full((16,), j, jnp.int32)
        r = plsc.load_gather(rows_v, [jv]) - n0

        x0a = plsc.load_gather(x_v, [jv, iota])
        x0b = plsc.load_gather(x_v, [jv, iota + 16])
        x1 = [plsc.load_gather(x_v, [jv, 3 * iota + (32 + c)]) for c in range(3)]

        y1 = [plsc.load_gather(w_v, [jv, jnp.full((16,), 112 + c, jnp.int32)])
              for c in range(3)]

        w1a = plsc.load_gather(w_v, [jv, iota])
        w1b = plsc.load_gather(w_v, [jv, iota + 16])
        w2 = plsc.load_gather(w_v, [jv, iota + 32])
        w3a = plsc.load_gather(w_v, [jv, iota + 48])
        w3b = plsc.load_gather(w_v, [jv, iota + 64])
        w4 = plsc.load_gather(w_v, [jv, iota + 80])
        w5 = plsc.load_gather(w_v, [jv, iota + 96])

        plsc.addupdate_scatter(acc_v, [r, c_o1a], w1a * x0a)
        plsc.addupdate_scatter(acc_v, [r, c_o1b], w1b * x0b)
        dot = x1[0] * y1[0] + x1[1] * y1[1] + x1[2] * y1[2]
        plsc.addupdate_scatter(acc_v, [r, c_o2], w2 * dot)
        t3a = w3a * x0a
        t3b = w3b * x0b
        for c in range(3):
            plsc.addupdate_scatter(acc_v, [r, c_o3a[c]], t3a * y1[c])
            plsc.addupdate_scatter(acc_v, [r, c_o3b[c]], t3b * y1[c])
        for c in range(3):
            plsc.addupdate_scatter(acc_v, [r, c_o4[c]], w4 * x1[c])
        cr = [x1[1] * y1[2] - x1[2] * y1[1],
              x1[2] * y1[0] - x1[0] * y1[2],
              x1[0] * y1[1] - x1[1] * y1[0]]
        for c in range(3):
            plsc.addupdate_scatter(acc_v, [r, c_o5[c]], w5 * cr[c])
        return carry

    def zero_body(i, _):
        for k in range(OUT_PAD // 16):
            acc_v[i, pl.ds(16 * k, 16)] = zero16
        return 0

    def task_body(i, _):
        t = wid + i * NW
        n0 = t * NB
        ev = offs_v[pl.ds(t, 16)]
        e0 = ev[0]
        e1 = ev[1]
        e0a = (e0 // 8) * 8
        nch = (e1 - e0a + CH - 1) // CH

        def base_of(c):
            return jnp.minimum(e0a + c * CH, N_EDGES - CH)

        def startA(k, c):
            b = base_of(c)
            pltpu.async_copy(rows_hbm.at[pl.ds(b, CH)], rows_b[k], semA.at[k])
            pltpu.async_copy(cols_hbm.at[pl.ds(b, CH)], cols_b[k], semA.at[k])
            pltpu.async_copy(w_hbm.at[pl.ds(b, CH)], w_b[k], semA.at[k])

        def waitA(k, c):
            b = base_of(c)
            pltpu.make_async_copy(
                rows_hbm.at[pl.ds(b, CH)], rows_b[k], semA.at[k]).wait()
            pltpu.make_async_copy(
                cols_hbm.at[pl.ds(b, CH)], cols_b[k], semA.at[k]).wait()
            pltpu.make_async_copy(
                w_hbm.at[pl.ds(b, CH)], w_b[k], semA.at[k]).wait()

        def startB(k):
            pltpu.async_copy(x_hbm.at[cols_b[k]], x_b[k], semB.at[k])

        def waitB(k):
            pltpu.make_async_copy(x_hbm.at[cols_b[k]], x_b[k], semB.at[k]).wait()

        @pl.when(nch > 0)
        def _():
            startA(0, 0)

        @pl.when(nch > 1)
        def _():
            startA(1, 1)

        lax.fori_loop(0, NB, zero_body, 0, unroll=False)

        @pl.when(nch > 0)
        def _():
            waitA(0, 0)
            startB(0)

        def group_body(g, _):
            for k in range(3):
                c = 3 * g + k

                @pl.when(c < nch)
                def _(c=c, k=k):
                    @pl.when(c + 1 < nch)
                    def _():
                        waitA((k + 1) % 3, c + 1)
                        startB((k + 1) % 3)

                    @pl.when(c + 2 < nch)
                    def _():
                        startA((k + 2) % 3, c + 2)

                    waitB(k)
                    bnom = e0a + c * CH
                    b = base_of(c)
                    jlo = jnp.maximum(e0, bnom) - b
                    jhi = jnp.minimum(e1, bnom + CH) - b
                    lax.fori_loop(jlo, jhi, make_edge_body(k), n0,
                                  unroll=False)
            return 0

        lax.fori_loop(0, (nch + 2) // 3, group_body, 0, unroll=False)
        pltpu.sync_copy(acc_v, z_hbm.at[pl.ds(n0, NB)])
        return 0

    ntasks = (T_TASKS - wid + NW - 1) // NW
    lax.fori_loop(0, ntasks, task_body, 0, unroll=False)


def _eye(n):
    return jnp.float32(
        lax.broadcasted_iota(jnp.int32, (n, n), 0)
        == lax.broadcasted_iota(jnp.int32, (n, n), 1))


_PACK_E = 6400


def _packw_body(wt_ref, yt_ref, o_ref):
    wt = wt_ref[...].T
    yt = yt_ref[...].T
    y0 = yt[:, 0:1]
    o_ref[...] = jnp.concatenate(
        [wt[:, 0:32] * y0,
         wt[:, 32:48] * INV_SQRT3,
         wt[:, 48:80],
         wt[:, 80:96] * y0,
         wt[:, 96:112] * INV_SQRT2,
         yt[:, 1:4],
         jnp.zeros((_PACK_E, REC - W_NUMEL - 3), jnp.float32)], axis=1)


_packw = pl.pallas_call(
    _packw_body,
    grid=(N_EDGES // _PACK_E,),
    in_specs=[
        pl.BlockSpec((W_NUMEL, _PACK_E), lambda i: (0, i)),
        pl.BlockSpec((4, _PACK_E), lambda i: (0, i)),
    ],
    out_specs=pl.BlockSpec((_PACK_E, REC), lambda i: (i, 0)),
    out_shape=jax.ShapeDtypeStruct((N_EDGES, REC), jnp.float32),
)

_UNPACK_N = 2176


def _unpack_body(z_ref, o_ref):
    o_ref[...] = z_ref[:, :OUT_DIM].T


_unpack = pl.pallas_call(
    _unpack_body,
    grid=(23,),
    in_specs=[pl.BlockSpec((_UNPACK_N, OUT_PAD), lambda i: (i, 0))],
    out_specs=pl.BlockSpec((OUT_DIM, _UNPACK_N), lambda i: (0, i)),
    out_shape=jax.ShapeDtypeStruct((OUT_DIM, N_NODES), jnp.float32),
)


_XPACK_N = 6400
_N_XPAD = 51200


def _packx_body(xt_ref, o_ref):
    o_ref[...] = jnp.concatenate(
        [xt_ref[...].T, jnp.zeros((_XPACK_N, XREC - IN1_DIM), jnp.float32)],
        axis=1)


_packx = pl.pallas_call(
    _packx_body,
    grid=(_N_XPAD // _XPACK_N,),
    in_specs=[pl.BlockSpec((IN1_DIM, _XPACK_N), lambda i: (0, i))],
    out_specs=pl.BlockSpec((_XPACK_N, XREC), lambda i: (i, 0)),
    out_shape=jax.ShapeDtypeStruct((_N_XPAD, XREC), jnp.float32),
)


@jax.jit
def _tp_conv(X, Y, W, rows, cols):
    x_p = _packx(X.T)
    w_p = _packw(W.T, Y.T)
    bounds = jnp.arange(0, NOFF, dtype=jnp.int32) * NB
    offs = jnp.searchsorted(rows, bounds, side="left").astype(jnp.int32)
    offs = jnp.minimum(offs, N_EDGES)

    mesh = plsc.VectorSubcoreMesh(core_axis_name="c", subcore_axis_name="s")
    run = pl.kernel(
        _sc_body,
        out_type=jax.ShapeDtypeStruct((N_NODES, OUT_PAD), jnp.float32),
        mesh=mesh,
        compiler_params=pltpu.CompilerParams(
            needs_layout_passes=False, use_tc_tiling_on_sc=False),
        scratch_types=(
            [pltpu.VMEM((NB, OUT_PAD), jnp.float32)]
            + [pltpu.VMEM((CH,), jnp.int32)] * 3
            + [pltpu.VMEM((CH,), jnp.int32)] * 3
            + [pltpu.VMEM((CH, REC), jnp.float32)] * 3
            + [pltpu.VMEM((CH, XREC), jnp.float32)] * 3
            + [pltpu.VMEM((NOFF,), jnp.int32),
               pltpu.SemaphoreType.DMA((3,)),
               pltpu.SemaphoreType.DMA((3,))]
        ),
    )
    z = run(x_p, rows, cols, w_p, offs)
    return _unpack(z).T


def kernel(X, Y, W, rows, cols):
    return _tp_conv(X, Y, W, rows, cols)


# speedup vs baseline: 7.5591x; 1.0000x over previous
import jax
import jax.numpy as jnp
from jax import lax
from jax.experimental import pallas as pl
from jax.experimental.pallas import tpu as pltpu
from jax.experimental.pallas import tpu_sc as plsc

N_NODES = 50000
N_EDGES = 800000
IN1_DIM = 80
W_NUMEL = 112
OUT_DIM = 240
OUT_PAD = 256
XREC = 128
REC = 128

NB = 80
CH = 128
T_TASKS = N_NODES // NB
NOFF = 640

INV_SQRT3 = 1.0 / (3.0 ** 0.5)
INV_SQRT2 = 1.0 / (2.0 ** 0.5)

NC = 2
NS = 16
NW = NC * NS


def _sc_body(x_hbm, rows_hbm, cols_hbm, w_hbm, offs_hbm, z_hbm,
             acc_v,
             rows_b0, rows_b1, rows_b2,
             cols_b0, cols_b1, cols_b2,
             w_b0, w_b1, w_b2,
             x_b0, x_b1, x_b2,
             offs_v, semA, semB):
    rows_b = (rows_b0, rows_b1, rows_b2)
    cols_b = (cols_b0, cols_b1, cols_b2)
    w_b = (w_b0, w_b1, w_b2)
    x_b = (x_b0, x_b1, x_b2)
    wid = lax.axis_index("s") * NC + lax.axis_index("c")

    pltpu.sync_copy(offs_hbm, offs_v)

    iota = lax.iota(jnp.int32, 16)
    zero16 = jnp.zeros((16,), jnp.float32)
    c_o1a = iota
    c_o1b = iota + 16
    c_o2 = iota + 32
    c_o3a = [3 * iota + (48 + c) for c in range(3)]
    c_o3b = [3 * iota + (96 + c) for c in range(3)]
    c_o4 = [3 * iota + (144 + c) for c in range(3)]
    c_o5 = [3 * iota + (192 + c) for c in range(3)]

    def make_edge_body(k):
        rows_v, w_v, x_v = rows_b[k], w_b[k], x_b[k]
        return lambda j, carry: edge_step(j, carry, rows_v, w_v, x_v)

    def edge_step(j, carry, rows_v, w_v, x_v):
        n0 = carry
        jv = jnp.full((16,), j, jnp.int32)
        r = plsc.load_gather(rows_v, [jv]) - n0

        x0a = plsc.load_gather(x_v, [jv, iota])
        x0b = plsc.load_gather(x_v, [jv, iota + 16])
        x1 = [plsc.load_gather(x_v, [jv, 3 * iota + (32 + c)]) for c in range(3)]

        y1 = [plsc.load_gather(w_v, [jv, jnp.full((16,), 112 + c, jnp.int32)])
              for c in range(3)]

        w1a = plsc.load_gather(w_v, [jv, iota])
        w1b = plsc.load_gather(w_v, [jv, iota + 16])
        w2 = plsc.load_gather(w_v, [jv, iota + 32])
        w3a = plsc.load_gather(w_v, [jv, iota + 48])
        w3b = plsc.load_gather(w_v, [jv, iota + 64])
        w4 = plsc.load_gather(w_v, [jv, iota + 80])
        w5 = plsc.load_gather(w_v, [jv, iota + 96])

        plsc.addupdate_scatter(acc_v, [r, c_o1a], w1a * x0a)
        plsc.addupdate_scatter(acc_v, [r, c_o1b], w1b * x0b)
        dot = x1[0] * y1[0] + x1[1] * y1[1] + x1[2] * y1[2]
        plsc.addupdate_scatter(acc_v, [r, c_o2], w2 * dot)
        t3a = w3a * x0a
        t3b = w3b * x0b
        for c in range(3):
            plsc.addupdate_scatter(acc_v, [r, c_o3a[c]], t3a * y1[c])
            plsc.addupdate_scatter(acc_v, [r, c_o3b[c]], t3b * y1[c])
        for c in range(3):
            plsc.addupdate_scatter(acc_v, [r, c_o4[c]], w4 * x1[c])
        cr = [x1[1] * y1[2] - x1[2] * y1[1],
              x1[2] * y1[0] - x1[0] * y1[2],
              x1[0] * y1[1] - x1[1] * y1[0]]
        for c in range(3):
            plsc.addupdate_scatter(acc_v, [r, c_o5[c]], w5 * cr[c])
        return carry

    def zero_body(i, _):
        for k in range(OUT_PAD // 16):
            acc_v[i, pl.ds(16 * k, 16)] = zero16
        return 0

    def task_body(i, _):
        t = wid + i * NW
        n0 = t * NB
        ev = offs_v[pl.ds(t, 16)]
        e0 = ev[0]
        e1 = ev[1]
        e0a = (e0 // 8) * 8
        nch = (e1 - e0a + CH - 1) // CH

        def base_of(c):
            return jnp.minimum(e0a + c * CH, N_EDGES - CH)

        def startA(k, c):
            b = base_of(c)
            pltpu.async_copy(rows_hbm.at[pl.ds(b, CH)], rows_b[k], semA.at[k])
            pltpu.async_copy(cols_hbm.at[pl.ds(b, CH)], cols_b[k], semA.at[k])
            pltpu.async_copy(w_hbm.at[pl.ds(b, CH)], w_b[k], semA.at[k])

        def waitA(k, c):
            b = base_of(c)
            pltpu.make_async_copy(
                rows_hbm.at[pl.ds(b, CH)], rows_b[k], semA.at[k]).wait()
            pltpu.make_async_copy(
                cols_hbm.at[pl.ds(b, CH)], cols_b[k], semA.at[k]).wait()
            pltpu.make_async_copy(
                w_hbm.at[pl.ds(b, CH)], w_b[k], semA.at[k]).wait()

        def startB(k):
            pltpu.async_copy(x_hbm.at[cols_b[k]], x_b[k], semB.at[k])

        def waitB(k):
            pltpu.make_async_copy(x_hbm.at[cols_b[k]], x_b[k], semB.at[k]).wait()

        @pl.when(nch > 0)
        def _():
            startA(0, 0)

        @pl.when(nch > 1)
        def _():
            startA(1, 1)

        lax.fori_loop(0, NB, zero_body, 0, unroll=False)

        @pl.when(nch > 0)
        def _():
            waitA(0, 0)
            startB(0)

        def group_body(g, _):
            for k in range(3):
                c = 3 * g + k

                @pl.when(c < nch)
                def _(c=c, k=k):
                    @pl.when(c + 1 < nch)
                    def _():
                        waitA((k + 1) % 3, c + 1)
                        startB((k + 1) % 3)

                    @pl.when(c + 2 < nch)
                    def _():
                        startA((k + 2) % 3, c + 2)

                    waitB(k)
                    bnom = e0a + c * CH
                    b = base_of(c)
                    jlo = jnp.maximum(e0, bnom) - b
                    jhi = jnp.minimum(e1, bnom + CH) - b
                    lax.fori_loop(jlo, jhi, make_edge_body(k), n0,
                                  unroll=False)
            return 0

        lax.fori_loop(0, (nch + 2) // 3, group_body, 0, unroll=False)
        pltpu.sync_copy(acc_v, z_hbm.at[pl.ds(n0, NB)])
        return 0

    ntasks = (T_TASKS - wid + NW - 1) // NW
    lax.fori_loop(0, ntasks, task_body, 0, unroll=False)


_PACK_E = 6400


def _packw_body(wt_ref, yt_ref, o_ref):
    wt = wt_ref[...].T
    yt = yt_ref[...].T
    y0 = yt[:, 0:1]
    o_ref[...] = jnp.concatenate(
        [wt[:, 0:32] * y0,
         wt[:, 32:48] * INV_SQRT3,
         wt[:, 48:80],
         wt[:, 80:96] * y0,
         wt[:, 96:112] * INV_SQRT2,
         yt[:, 1:4],
         jnp.zeros((_PACK_E, REC - W_NUMEL - 3), jnp.float32)], axis=1)


_packw = pl.pallas_call(
    _packw_body,
    grid=(N_EDGES // _PACK_E,),
    in_specs=[
        pl.BlockSpec((W_NUMEL, _PACK_E), lambda i: (0, i)),
        pl.BlockSpec((4, _PACK_E), lambda i: (0, i)),
    ],
    out_specs=pl.BlockSpec((_PACK_E, REC), lambda i: (i, 0)),
    out_shape=jax.ShapeDtypeStruct((N_EDGES, REC), jnp.float32),
)

_UNPACK_N = 2176


def _unpack_body(z_ref, o_ref):
    o_ref[...] = z_ref[:, :OUT_DIM].T


_unpack = pl.pallas_call(
    _unpack_body,
    grid=(23,),
    in_specs=[pl.BlockSpec((_UNPACK_N, OUT_PAD), lambda i: (i, 0))],
    out_specs=pl.BlockSpec((OUT_DIM, _UNPACK_N), lambda i: (0, i)),
    out_shape=jax.ShapeDtypeStruct((OUT_DIM, N_NODES), jnp.float32),
)


_XPACK_N = 6400
_N_XPAD = 51200


def _packx_body(xt_ref, o_ref):
    o_ref[...] = jnp.concatenate(
        [xt_ref[...].T, jnp.zeros((_XPACK_N, XREC - IN1_DIM), jnp.float32)],
        axis=1)


_packx = pl.pallas_call(
    _packx_body,
    grid=(_N_XPAD // _XPACK_N,),
    in_specs=[pl.BlockSpec((IN1_DIM, _XPACK_N), lambda i: (0, i))],
    out_specs=pl.BlockSpec((_XPACK_N, XREC), lambda i: (i, 0)),
    out_shape=jax.ShapeDtypeStruct((_N_XPAD, XREC), jnp.float32),
)


@jax.jit
def _tp_conv(X, Y, W, rows, cols):
    x_p = _packx(X.T)
    w_p = _packw(W.T, Y.T)
    bounds = jnp.arange(0, NOFF, dtype=jnp.int32) * NB
    offs = jnp.searchsorted(rows, bounds, side="left").astype(jnp.int32)
    offs = jnp.minimum(offs, N_EDGES)

    mesh = plsc.VectorSubcoreMesh(core_axis_name="c", subcore_axis_name="s")
    run = pl.kernel(
        _sc_body,
        out_type=jax.ShapeDtypeStruct((N_NODES, OUT_PAD), jnp.float32),
        mesh=mesh,
        compiler_params=pltpu.CompilerParams(
            needs_layout_passes=False, use_tc_tiling_on_sc=False),
        scratch_types=(
            [pltpu.VMEM((NB, OUT_PAD), jnp.float32)]
            + [pltpu.VMEM((CH,), jnp.int32)] * 3
            + [pltpu.VMEM((CH,), jnp.int32)] * 3
            + [pltpu.VMEM((CH, REC), jnp.float32)] * 3
            + [pltpu.VMEM((CH, XREC), jnp.float32)] * 3
            + [pltpu.VMEM((NOFF,), jnp.int32),
               pltpu.SemaphoreType.DMA((3,)),
               pltpu.SemaphoreType.DMA((3,))]
        ),
    )
    z = run(x_p, rows, cols, w_p, offs)
    return _unpack(z).T


def kernel(X, Y, W, rows, cols):
    return _tp_conv(X, Y, W, rows, cols)
